# baseline (speedup 1.0000x reference)
import functools
import math

import jax
import jax.numpy as jnp
from jax import lax
from jax.experimental import pallas as pl
from jax.experimental.pallas import tpu as pltpu
from jax.experimental.pallas import tpu_sc as plsc

_VOCAB = 1_000_000
_D = 64
_BATCH = 4096
_SEQ = 200

_NC = 2
_NS = 16
_NW = _NC * _NS
_SPW = _BATCH // _NW
_SLAB = 8
_L = 16
_NGRP = _SEQ // _L + 1

_GDN = lax.GatherDimensionNumbers(
    offset_dims=(), collapsed_slice_dims=(0,), start_index_map=(0,))


def _vsplat(vec, lane):
    idx = jnp.full((_L, 1), lane, dtype=jnp.int32)
    return lax.gather(vec, idx, _GDN, (1,),
                      mode=lax.GatherScatterMode.PROMISE_IN_BOUNDS)


def _make_pe(seq_len, emb_dim):
    position = jnp.arange(seq_len, dtype=jnp.float32)[:, None]
    div_term = jnp.exp(
        jnp.arange(0, emb_dim, 2, dtype=jnp.float32)
        * (-math.log(10000.0) / emb_dim))
    pe = jnp.zeros((seq_len, emb_dim), dtype=jnp.float32)
    pe = pe.at[:, 0::2].set(jnp.sin(position * div_term))
    pe = pe.at[:, 1::2].set(jnp.cos(position * div_term))
    return pe


def _emb_body(x_hbm, tab2_hbm, pe2_hbm, out_hbm,
              idx_v, idx2_v, buf_a, buf_b, obuf_a, obuf_b, pe_v, gsem, wsem_a, wsem_b):
    wid = lax.axis_index("s") * _NC + lax.axis_index("c")
    seq0_w = wid * _SPW
    iota = lax.iota(jnp.int32, _L)

    pltpu.sync_copy(pe2_hbm, pe_v)

    def compute(t, buf, obuf):
        @plsc.parallel_loop(0, _NGRP)
        def grp_body(g):
            rb = jnp.minimum(g * _L, _SEQ - _L)
            raw = idx_v[t, pl.ds(rb, _L)]
            par = lax.shift_left(
                lax.bitwise_and(raw, jnp.int32(1)), jnp.int32(6))
            for rr in range(_L):
                parv = _vsplat(par, rr)
                row = rb + rr
                rowv = jnp.broadcast_to(row, (_L,))
                colv = parv + iota
                ph = lax.shift_right_logical(row, 1)
                pc = lax.shift_left(
                    lax.bitwise_and(row, jnp.int32(1)), jnp.int32(6))
                for dq in range(_D // _L):
                    vals = plsc.load_gather(
                        buf, [rowv, colv + jnp.int32(dq * _L)])
                    pev = pe_v[ph, pl.ds(pc + dq * _L, _L)]
                    obuf[row, pl.ds(dq * _L, _L)] = vals + pev

    def slab_body(sl, carry):
        seq0 = seq0_w + sl * _SLAB
        pltpu.sync_copy(x_hbm.at[pl.ds(seq0, _SLAB)], idx_v)

        @plsc.parallel_loop(0, _SLAB)
        def halve_body(i):
            for g in range(_NGRP):
                cb = min(g * _L, _SEQ - _L)
                idx2_v[i, pl.ds(cb, _L)] = lax.shift_right_logical(
                    idx_v[i, pl.ds(cb, _L)], 1)

        def gath(t, buf):
            return pltpu.async_copy(
                tab2_hbm.at[idx2_v.at[t, pl.ds(0, 128)]],
                buf.at[pl.ds(0, 128)], gsem), pltpu.async_copy(
                tab2_hbm.at[idx2_v.at[t, pl.ds(128, 72)]],
                buf.at[pl.ds(128, 72)], gsem)

        def wwait(obuf, ws):
            pltpu.make_async_copy(obuf, out_hbm.at[seq0_w], ws).wait()

        def pair_body(j, c2):
            ta = 2 * j
            g1, g2 = gath(ta + 1, buf_b)

            @pl.when(jnp.logical_or(j > 0, sl > 0))
            def _():
                wwait(obuf_a, wsem_a)

            compute(ta, buf_a, obuf_a)
            pltpu.async_copy(obuf_a, out_hbm.at[seq0 + ta], wsem_a)
            g1.wait()
            g2.wait()

            @pl.when(j < _SLAB // 2 - 1)
            def _():
                gath(ta + 2, buf_a)

            @pl.when(jnp.logical_or(j > 0, sl > 0))
            def _():
                wwait(obuf_b, wsem_b)

            compute(ta + 1, buf_b, obuf_b)
            pltpu.async_copy(obuf_b, out_hbm.at[seq0 + ta + 1], wsem_b)

            @pl.when(j < _SLAB // 2 - 1)
            def _():
                pltpu.make_async_copy(
                    tab2_hbm.at[idx2_v.at[ta + 2, pl.ds(0, 128)]],
                    buf_a.at[pl.ds(0, 128)], gsem).wait()
                pltpu.make_async_copy(
                    tab2_hbm.at[idx2_v.at[ta + 2, pl.ds(128, 72)]],
                    buf_a.at[pl.ds(128, 72)], gsem).wait()
            return c2

        ga, gb = gath(0, buf_a)
        ga.wait()
        gb.wait()
        lax.fori_loop(0, _SLAB // 2, pair_body, 0)
        return carry

    lax.fori_loop(0, _SPW // _SLAB, slab_body, 0)
    pltpu.make_async_copy(obuf_a, out_hbm.at[seq0_w], wsem_a).wait()
    pltpu.make_async_copy(obuf_b, out_hbm.at[seq0_w], wsem_b).wait()


_emb_call = functools.partial(
    pl.kernel,
    out_type=jax.ShapeDtypeStruct((_BATCH, _SEQ, _D), jnp.float32),
    mesh=plsc.VectorSubcoreMesh(core_axis_name="c", subcore_axis_name="s"),
    scratch_types=[
        pltpu.VMEM((_SLAB, _SEQ), jnp.int32),
        pltpu.VMEM((_SLAB, _SEQ), jnp.int32),
        pltpu.VMEM((_SEQ, 128), jnp.float32),
        pltpu.VMEM((_SEQ, 128), jnp.float32),
        pltpu.VMEM((_SEQ, _D), jnp.float32),
        pltpu.VMEM((_SEQ, _D), jnp.float32),
        pltpu.VMEM((_SEQ // 2, 128), jnp.float32),
        pltpu.SemaphoreType.DMA,
        pltpu.SemaphoreType.DMA,
        pltpu.SemaphoreType.DMA,
    ],
    compiler_params=pltpu.CompilerParams(
        use_tc_tiling_on_sc=True, needs_layout_passes=False),
)(_emb_body)


@jax.jit
def kernel(x, table):
    pe2 = _make_pe(_SEQ, _D).reshape(_SEQ // 2, 128)
    tab2 = jnp.concatenate([table[0::2, :], table[1::2, :]], axis=1)
    return _emb_call(x.astype(jnp.int32), tab2, pe2)


# speedup vs baseline: 7.4637x; 7.4637x over previous
import functools
import math

import jax
import jax.numpy as jnp
from jax import lax
from jax.experimental import pallas as pl
from jax.experimental.pallas import tpu as pltpu
from jax.experimental.pallas import tpu_sc as plsc

_VOCAB = 1_000_000
_D = 64
_BATCH = 4096
_SEQ = 200

_NC = 2
_NS = 16
_NW = _NC * _NS
_SPW = _BATCH // _NW
_SLAB = 8
_L = 16
_NGRP = _SEQ // _L + 1

_GDN = lax.GatherDimensionNumbers(
    offset_dims=(), collapsed_slice_dims=(0,), start_index_map=(0,))


def _vsplat(vec, lane):
    idx = jnp.full((_L, 1), lane, dtype=jnp.int32)
    return lax.gather(vec, idx, _GDN, (1,),
                      mode=lax.GatherScatterMode.PROMISE_IN_BOUNDS)


def _make_pe(seq_len, emb_dim):
    position = jnp.arange(seq_len, dtype=jnp.float32)[:, None]
    div_term = jnp.exp(
        jnp.arange(0, emb_dim, 2, dtype=jnp.float32)
        * (-math.log(10000.0) / emb_dim))
    pe = jnp.zeros((seq_len, emb_dim), dtype=jnp.float32)
    pe = pe.at[:, 0::2].set(jnp.sin(position * div_term))
    pe = pe.at[:, 1::2].set(jnp.cos(position * div_term))
    return pe


def _emb_body(x_hbm, tab2_hbm, pe2_hbm, out_hbm,
              idx_v, idx2_v, buf_a, buf_b, obuf_a, obuf_b, pe_v, gsem, wsem_a, wsem_b):
    wid = lax.axis_index("s") * _NC + lax.axis_index("c")
    seq0_w = wid * _SPW
    iota = lax.iota(jnp.int32, _L)

    pltpu.sync_copy(pe2_hbm, pe_v)

    def compute(t, buf, obuf):
        @plsc.parallel_loop(0, _NGRP)
        def grp_body(g):
            rb = jnp.minimum(g * _L, _SEQ - _L)
            raw = idx_v[t, pl.ds(rb, _L)]
            par = lax.shift_left(
                lax.bitwise_and(raw, jnp.int32(1)), jnp.int32(6))
            for rr in range(_L):
                parv = _vsplat(par, rr)
                row = rb + rr
                rowv = jnp.broadcast_to(row, (_L,))
                colv = parv + iota
                ph = lax.shift_right_logical(row, 1)
                pc = lax.shift_left(
                    lax.bitwise_and(row, jnp.int32(1)), jnp.int32(6))
                for dq in range(_D // _L):
                    vals = plsc.load_gather(
                        buf, [rowv, colv + jnp.int32(dq * _L)])
                    pev = pe_v[ph, pl.ds(pc + dq * _L, _L)]
                    obuf[row, pl.ds(dq * _L, _L)] = vals + pev

    def slab_body(sl, carry):
        seq0 = seq0_w + sl * _SLAB
        pltpu.sync_copy(x_hbm.at[pl.ds(seq0, _SLAB)], idx_v)

        @plsc.parallel_loop(0, _SLAB)
        def halve_body(i):
            for g in range(_NGRP):
                cb = min(g * _L, _SEQ - _L)
                idx2_v[i, pl.ds(cb, _L)] = lax.shift_right_logical(
                    idx_v[i, pl.ds(cb, _L)], 1)

        def gath(t, buf):
            return pltpu.async_copy(
                tab2_hbm.at[idx2_v.at[t, pl.ds(0, 128)]],
                buf.at[pl.ds(0, 128)], gsem), pltpu.async_copy(
                tab2_hbm.at[idx2_v.at[t, pl.ds(128, 72)]],
                buf.at[pl.ds(128, 72)], gsem)

        def wwait(obuf, ws):
            pltpu.make_async_copy(obuf, out_hbm.at[seq0_w], ws).wait()

        def pair_body(j, c2):
            ta = 2 * j
            g1, g2 = gath(ta + 1, buf_b)

            @pl.when(jnp.logical_or(j > 0, sl > 0))
            def _():
                wwait(obuf_a, wsem_a)

            compute(ta, buf_a, obuf_a)
            pltpu.async_copy(obuf_a, out_hbm.at[seq0 + ta], wsem_a)
            g1.wait()
            g2.wait()

            @pl.when(j < _SLAB // 2 - 1)
            def _():
                gath(ta + 2, buf_a)

            @pl.when(jnp.logical_or(j > 0, sl > 0))
            def _():
                wwait(obuf_b, wsem_b)

            compute(ta + 1, buf_b, obuf_b)
            pltpu.async_copy(obuf_b, out_hbm.at[seq0 + ta + 1], wsem_b)

            @pl.when(j < _SLAB // 2 - 1)
            def _():
                pltpu.make_async_copy(
                    tab2_hbm.at[idx2_v.at[ta + 2, pl.ds(0, 128)]],
                    buf_a.at[pl.ds(0, 128)], gsem).wait()
                pltpu.make_async_copy(
                    tab2_hbm.at[idx2_v.at[ta + 2, pl.ds(128, 72)]],
                    buf_a.at[pl.ds(128, 72)], gsem).wait()
            return c2

        ga, gb = gath(0, buf_a)
        ga.wait()
        gb.wait()
        lax.fori_loop(0, _SLAB // 2, pair_body, 0)
        return carry

    lax.fori_loop(0, _SPW // _SLAB, slab_body, 0)
    pltpu.make_async_copy(obuf_a, out_hbm.at[seq0_w], wsem_a).wait()
    pltpu.make_async_copy(obuf_b, out_hbm.at[seq0_w], wsem_b).wait()


_emb_call = functools.partial(
    pl.kernel,
    out_type=jax.ShapeDtypeStruct((_BATCH, _SEQ, _D), jnp.float32),
    mesh=plsc.VectorSubcoreMesh(core_axis_name="c", subcore_axis_name="s"),
    scratch_types=[
        pltpu.VMEM((_SLAB, _SEQ), jnp.int32),
        pltpu.VMEM((_SLAB, _SEQ), jnp.int32),
        pltpu.VMEM((_SEQ, 128), jnp.float32),
        pltpu.VMEM((_SEQ, 128), jnp.float32),
        pltpu.VMEM((_SEQ, _D), jnp.float32),
        pltpu.VMEM((_SEQ, _D), jnp.float32),
        pltpu.VMEM((_SEQ // 2, 128), jnp.float32),
        pltpu.SemaphoreType.DMA,
        pltpu.SemaphoreType.DMA,
        pltpu.SemaphoreType.DMA,
    ],
    compiler_params=pltpu.CompilerParams(
        use_tc_tiling_on_sc=True, needs_layout_passes=False),
)(_emb_body)


@jax.jit
def kernel(x, table):
    pe2 = _make_pe(_SEQ, _D).reshape(_SEQ // 2, 128)
    tab2 = table.reshape(_VOCAB // 2, 128)
    return _emb_call(x.astype(jnp.int32), tab2, pe2)
